# Initial kernel scaffold; baseline (speedup 1.0000x reference)
import jax
import jax.numpy as jnp
from jax.experimental import pallas as pl


def kernel(predictions, targets, bins, bin_weights):
    raise NotImplementedError("write your pallas kernel here")


# speedup vs baseline: 6.3288x; 6.3288x over previous
import jax
import jax.numpy as jnp
from jax.experimental import pallas as pl
from jax.experimental.pallas import tpu as pltpu

_ROWS = 16384
_COLS = 200
_BLOCK_ROWS = 2048
_GRID = _ROWS // _BLOCK_ROWS
_NBINS = 10


def _wmse_block(p_ref, t_ref, bins_ref, bw_ref, out_ref):
    p = p_ref[...]
    t = t_ref[...]
    l = (p - t) * (p - t)
    w = jnp.full_like(t, bw_ref[0])
    for j in range(1, _NBINS):
        w = jnp.where(t >= bins_ref[j], bw_ref[j], w)

    @pl.when(pl.program_id(0) == 0)
    def _init():
        out_ref[0, 0] = 0.0

    out_ref[0, 0] += jnp.sum(l * w)


def kernel(predictions, targets, bins, bin_weights):
    bw_scaled = bin_weights * (100.0 / (_ROWS * _COLS))
    out = pl.pallas_call(
        _wmse_block,
        grid=(_GRID,),
        in_specs=[
            pl.BlockSpec((_BLOCK_ROWS, _COLS), lambda i: (i, 0)),
            pl.BlockSpec((_BLOCK_ROWS, _COLS), lambda i: (i, 0)),
            pl.BlockSpec(memory_space=pltpu.SMEM),
            pl.BlockSpec(memory_space=pltpu.SMEM),
        ],
        out_specs=pl.BlockSpec((1, 1), lambda i: (0, 0), memory_space=pltpu.SMEM),
        out_shape=jax.ShapeDtypeStruct((1, 1), jnp.float32),
    )(predictions, targets, bins, bw_scaled)
    return out[0, 0]
